# baseline (speedup 1.0000x reference)
import functools

import jax
import jax.numpy as jnp
from jax import lax
from jax.experimental import pallas as pl
from jax.experimental.pallas import tpu as pltpu
from jax.experimental.pallas import tpu_sc as plsc

N = 10000
E = 320000
F = 128
H = 64
C = 16

N1 = 10240
NSC = 2
NTILE = 16
NW = NSC * NTILE
TPW = N1 // NTILE
CHUNK = 80
CHUNKS = 125
EPT = CHUNK * CHUNKS

_MESH = plsc.VectorSubcoreMesh(core_axis_name="c", subcore_axis_name="s")
_SC_UNTILED = pltpu.CompilerParams(use_tc_tiling_on_sc=False)
_SC_TILED = pltpu.CompilerParams(use_tc_tiling_on_sc=True)


def _make_prop(D):
    @functools.partial(
        pl.kernel,
        out_type=jax.ShapeDtypeStruct((NSC, N1, D), jnp.float32),
        mesh=_MESH,
        scratch_types=[
            pltpu.VMEM_SHARED((N1, D), jnp.float32),
            pltpu.VMEM((4, CHUNK), jnp.int32),
            pltpu.VMEM((4, CHUNK), jnp.int32),
            pltpu.VMEM((4, CHUNK, D), jnp.float32),
            pltpu.SemaphoreType.DMA,
            pltpu.SemaphoreType.DMA,
            pltpu.SemaphoreType.DMA,
            pltpu.SemaphoreType.DMA,
            pltpu.SemaphoreType.DMA,
            pltpu.SemaphoreType.DMA,
            pltpu.SemaphoreType.DMA,
            pltpu.SemaphoreType.DMA,
        ],
        compiler_params=_SC_TILED if D == F else _SC_UNTILED,
    )
    def prop(hs_hbm, src_hbm, dst_hbm, out_hbm, acc, sidx, didx, rows,
             g0, g1, g2, g3, i0, i1, i2, i3):
        gsem = (g0, g1, g2, g3)
        isem = (i0, i1, i2, i3)
        c = lax.axis_index("c")
        s = lax.axis_index("s")
        wid = c * NTILE + s
        base = wid * EPT
        r0 = s * TPW
        pltpu.sync_copy(hs_hbm.at[pl.ds(r0, TPW)], acc.at[pl.ds(r0, TPW)])
        plsc.subcore_barrier()

        def _fire_i(j, i):
            pltpu.async_copy(src_hbm.at[pl.ds(base + j * CHUNK, CHUNK)],
                             sidx.at[i], isem[i])
            pltpu.async_copy(dst_hbm.at[pl.ds(base + j * CHUNK, CHUNK)],
                             didx.at[i], isem[i])

        def _wait_i(j, i):
            pltpu.make_async_copy(src_hbm.at[pl.ds(base + j * CHUNK, CHUNK)],
                                  sidx.at[i], isem[i]).wait()
            pltpu.make_async_copy(dst_hbm.at[pl.ds(base + j * CHUNK, CHUNK)],
                                  didx.at[i], isem[i]).wait()

        def _fire_g(i):
            pltpu.async_copy(hs_hbm.at[sidx.at[i]], rows.at[i], gsem[i])

        def _wait_g(i):
            pltpu.make_async_copy(hs_hbm.at[sidx.at[i]], rows.at[i], gsem[i]).wait()

        def _scatter(i):
            pltpu.sync_copy(rows.at[i], acc.at[didx.at[i]], add=True)

        for i in range(4):
            _fire_i(i, i)
        for i in range(3):
            _wait_i(i, i)
            _fire_g(i)

        @pl.loop(0, CHUNKS // 4 - 1)
        def _(k):
            for i in range(4):
                j = k * 4 + i
                _wait_g(i)
                _scatter(i)
                _fire_i(j + 4, i)
                _wait_i(j + 3, (i + 3) % 4)
                _fire_g((i + 3) % 4)

        _wait_g(0)
        _scatter(0)
        _fire_i(124, 0)
        _wait_i(123, 3)
        _fire_g(3)
        _wait_g(1)
        _scatter(1)
        _wait_i(124, 0)
        _fire_g(0)
        _wait_g(2)
        _scatter(2)
        _wait_g(3)
        _scatter(3)
        _wait_g(0)
        _scatter(0)

        plsc.subcore_barrier()
        pltpu.sync_copy(acc.at[pl.ds(r0, TPW)], out_hbm.at[c].at[pl.ds(r0, TPW)])

    return prop


@functools.partial(
    pl.kernel,
    out_type=jax.ShapeDtypeStruct((NSC, N1, 1), jnp.float32),
    mesh=_MESH,
    scratch_types=[
        pltpu.VMEM_SHARED((N1, 1), jnp.float32),
        pltpu.VMEM((CHUNKS, CHUNK), jnp.int32),
        pltpu.VMEM((CHUNK, 1), jnp.float32),
        pltpu.SemaphoreType.DMA,
    ],
    compiler_params=_SC_UNTILED,
)
def _deg_pass(ones_hbm, dst_hbm, out_hbm, acc, dst_v, ones_v, sem):
    c = lax.axis_index("c")
    s = lax.axis_index("s")
    wid = c * NTILE + s
    r0 = s * TPW
    pltpu.sync_copy(ones_hbm.at[pl.ds(r0, TPW)], acc.at[pl.ds(r0, TPW)])
    pltpu.sync_copy(ones_hbm.at[pl.ds(0, CHUNK)], ones_v)
    pltpu.sync_copy(dst_hbm.at[wid], dst_v)
    plsc.subcore_barrier()

    @pl.loop(0, CHUNKS)
    def _(j):
        pltpu.sync_copy(ones_v, acc.at[dst_v.at[j]], add=True)

    plsc.subcore_barrier()
    pltpu.sync_copy(acc.at[pl.ds(r0, TPW)], out_hbm.at[c].at[pl.ds(r0, TPW)])


_RB = 1280
_GRID = (N1 // _RB,)


def _rows_spec(D):
    return pl.BlockSpec((_RB, D), lambda i: (i, 0))


def _pair_spec(D):
    return pl.BlockSpec((NSC, _RB, D), lambda i: (0, i, 0))


def _full_spec(a, b):
    return pl.BlockSpec((a, b), lambda i: (0, 0))


def _dis_spec():
    return pl.BlockSpec((_RB, 1), lambda i: (i, 0))


def _dis_block(dis_ref):
    return dis_ref[...]


def _t0_body(d_ref, x_ref, dis_ref, hs_ref):
    i = pl.program_id(0)
    deg = d_ref[0] + d_ref[1] - 1.0
    row = i * _RB + lax.broadcasted_iota(jnp.int32, (_RB, 1), 0)
    dis = jnp.where(row < N, lax.rsqrt(deg), 0.0)
    dis_ref[...] = dis
    hs_ref[...] = x_ref[...] * dis


def _t0(d, x_pad):
    return pl.pallas_call(
        _t0_body,
        grid=_GRID,
        in_specs=[_pair_spec(1), _rows_spec(F)],
        out_specs=[_dis_spec(), _rows_spec(F)],
        out_shape=[
            jax.ShapeDtypeStruct((N1, 1), jnp.float32),
            jax.ShapeDtypeStruct((N1, F), jnp.float32),
        ],
    )(d, x_pad)


def _t1_body(s_ref, hs_ref, dis_ref, w1_ref, b1_ref, w2_ref, out_ref):
    dis = _dis_block(dis_ref)
    ax = (s_ref[0] + s_ref[1] - hs_ref[...]) * dis
    h1 = jax.nn.relu(
        jnp.dot(ax, w1_ref[...], preferred_element_type=jnp.float32)
        + b1_ref[...]
    )
    out_ref[...] = (
        jnp.dot(h1, w2_ref[...], preferred_element_type=jnp.float32) * dis
    )


def _t1(s, hs0, dis, W1, b1, W2):
    return pl.pallas_call(
        _t1_body,
        grid=_GRID,
        in_specs=[
            _pair_spec(F),
            _rows_spec(F),
            _dis_spec(),
            _full_spec(F, 3 * H),
            _full_spec(1, 3 * H),
            _full_spec(3 * H, 2 * H),
        ],
        out_specs=_rows_spec(2 * H),
        out_shape=jax.ShapeDtypeStruct((N1, 2 * H), jnp.float32),
    )(s, hs0, dis, W1, b1.reshape(1, -1), W2)


def _mid_body(s_ref, hs_ref, dis_ref, b_ref, w_ref, out_ref):
    dis = _dis_block(dis_ref)
    h = jax.nn.relu((s_ref[0] + s_ref[1] - hs_ref[...]) * dis + b_ref[...])
    out_ref[...] = (
        jnp.dot(h, w_ref[...], preferred_element_type=jnp.float32) * dis
    )


def _t_mid(s, hs, dis, b, Wn, Din, Dout):
    return pl.pallas_call(
        _mid_body,
        grid=_GRID,
        in_specs=[
            _pair_spec(Din),
            _rows_spec(Din),
            _dis_spec(),
            _full_spec(1, Din),
            _full_spec(Din, Dout),
        ],
        out_specs=_rows_spec(Dout),
        out_shape=jax.ShapeDtypeStruct((N1, Dout), jnp.float32),
    )(s, hs, dis, b.reshape(1, -1), Wn)


def _t4_body(s_ref, hs_ref, dis_ref, b_ref, out_ref):
    dis = _dis_block(dis_ref)
    out_ref[...] = (
        (s_ref[0] + s_ref[1] - hs_ref[...]) * dis + b_ref[...]
    )


def _t4(s, hs, dis, b):
    return pl.pallas_call(
        _t4_body,
        grid=_GRID,
        in_specs=[
            _pair_spec(C),
            _rows_spec(C),
            _dis_spec(),
            _full_spec(1, C),
        ],
        out_specs=_rows_spec(C),
        out_shape=jax.ShapeDtypeStruct((N1, C), jnp.float32),
    )(s, hs, dis, b.reshape(1, -1))


_prop128 = _make_prop(F)
_prop64 = _make_prop(H)
_prop16 = _make_prop(C)


def kernel(x, edge_index, W1, b1, W2, b2, W3, b3, W4, b4):
    src = edge_index[0]
    dst = edge_index[1]

    x_pad = jnp.pad(x, ((0, N1 - N), (0, 0)))
    ones = jnp.ones((N1, 1), jnp.float32)

    d = _deg_pass(ones, dst.reshape(NW, CHUNKS, CHUNK))
    dis, hs0 = _t0(d, x_pad)

    s1 = _prop128(hs0, src, dst)
    hs1 = _t1(s1, hs0, dis, W1, b1, W2)

    s2 = _prop128(hs1, src, dst)
    hs2 = _t_mid(s2, hs1, dis, b2, W3, 2 * H, H)

    s3 = _prop64(hs2, src, dst)
    hs3 = _t_mid(s3, hs2, dis, b3, W4, H, C)

    s4 = _prop16(hs3, src, dst)
    out = _t4(s4, hs3, dis, b4)
    return out[:N]


# speedup vs baseline: 1.0091x; 1.0091x over previous
import functools

import jax
import jax.numpy as jnp
from jax import lax
from jax.experimental import pallas as pl
from jax.experimental.pallas import tpu as pltpu
from jax.experimental.pallas import tpu_sc as plsc

N = 10000
E = 320000
F = 128
H = 64
C = 16

N1 = 10240
NSC = 2
NTILE = 16
NW = NSC * NTILE
TPW = N1 // NTILE
CHUNK = 80
CHUNKS = 125
EPT = CHUNK * CHUNKS

_MESH = plsc.VectorSubcoreMesh(core_axis_name="c", subcore_axis_name="s")
_SC_UNTILED = pltpu.CompilerParams(use_tc_tiling_on_sc=False)
_SC_TILED = pltpu.CompilerParams(use_tc_tiling_on_sc=True)


def _make_prop(D):
    @functools.partial(
        pl.kernel,
        out_type=jax.ShapeDtypeStruct((NSC, N1, D), jnp.float32),
        mesh=_MESH,
        scratch_types=[
            pltpu.VMEM_SHARED((N1, D), jnp.float32),
            pltpu.VMEM((4, CHUNK), jnp.int32),
            pltpu.VMEM((4, CHUNK), jnp.int32),
            pltpu.VMEM((4, CHUNK, D), jnp.float32),
            pltpu.SemaphoreType.DMA,
            pltpu.SemaphoreType.DMA,
            pltpu.SemaphoreType.DMA,
            pltpu.SemaphoreType.DMA,
            pltpu.SemaphoreType.DMA,
            pltpu.SemaphoreType.DMA,
            pltpu.SemaphoreType.DMA,
            pltpu.SemaphoreType.DMA,
        ],
        compiler_params=_SC_TILED if D == F else _SC_UNTILED,
    )
    def prop(hs_hbm, src_hbm, dst_hbm, out_hbm, acc, sidx, didx, rows,
             g0, g1, g2, g3, i0, i1, i2, i3):
        gsem = (g0, g1, g2, g3)
        isem = (i0, i1, i2, i3)
        c = lax.axis_index("c")
        s = lax.axis_index("s")
        wid = c * NTILE + s
        base = wid * EPT
        r0 = s * TPW
        pltpu.sync_copy(hs_hbm.at[pl.ds(r0, TPW)], acc.at[pl.ds(r0, TPW)])
        plsc.subcore_barrier()

        def _fire_i(j, i):
            pltpu.async_copy(src_hbm.at[pl.ds(base + j * CHUNK, CHUNK)],
                             sidx.at[i], isem[i])
            pltpu.async_copy(dst_hbm.at[pl.ds(base + j * CHUNK, CHUNK)],
                             didx.at[i], isem[i])

        def _wait_i(j, i):
            pltpu.make_async_copy(src_hbm.at[pl.ds(base + j * CHUNK, CHUNK)],
                                  sidx.at[i], isem[i]).wait()
            pltpu.make_async_copy(dst_hbm.at[pl.ds(base + j * CHUNK, CHUNK)],
                                  didx.at[i], isem[i]).wait()

        def _fire_g(i):
            pltpu.async_copy(hs_hbm.at[sidx.at[i]], rows.at[i], gsem[i])

        def _wait_g(i):
            pltpu.make_async_copy(hs_hbm.at[sidx.at[i]], rows.at[i], gsem[i]).wait()

        def _scatter(i):
            pltpu.sync_copy(rows.at[i], acc.at[didx.at[i]], add=True)

        for i in range(4):
            _fire_i(i, i)
        for i in range(3):
            _wait_i(i, i)
            _fire_g(i)

        @pl.loop(0, CHUNKS // 4 - 1)
        def _(k):
            for i in range(4):
                j = k * 4 + i
                _wait_g(i)
                _scatter(i)
                _fire_i(j + 4, i)
                _wait_i(j + 3, (i + 3) % 4)
                _fire_g((i + 3) % 4)

        _wait_g(0)
        _scatter(0)
        _fire_i(124, 0)
        _wait_i(123, 3)
        _fire_g(3)
        _wait_g(1)
        _scatter(1)
        _wait_i(124, 0)
        _fire_g(0)
        _wait_g(2)
        _scatter(2)
        _wait_g(3)
        _scatter(3)
        _wait_g(0)
        _scatter(0)

        plsc.subcore_barrier()
        pltpu.sync_copy(acc.at[pl.ds(r0, TPW)], out_hbm.at[c].at[pl.ds(r0, TPW)])

    return prop


@functools.partial(
    pl.kernel,
    out_type=jax.ShapeDtypeStruct((NSC, N1, 1), jnp.float32),
    mesh=_MESH,
    scratch_types=[
        pltpu.VMEM_SHARED((N1, 1), jnp.float32),
        pltpu.VMEM((CHUNKS, CHUNK), jnp.int32),
        pltpu.VMEM((CHUNK, 1), jnp.float32),
        pltpu.SemaphoreType.DMA,
        pltpu.SemaphoreType.DMA,
    ],
    compiler_params=_SC_UNTILED,
)
def _deg_pass(ones_hbm, dst_hbm, out_hbm, acc, dst_v, ones_v, s0, s1):
    c = lax.axis_index("c")
    s = lax.axis_index("s")
    wid = c * NTILE + s
    r0 = s * TPW
    pltpu.sync_copy(ones_hbm.at[pl.ds(r0, TPW)], acc.at[pl.ds(r0, TPW)])
    pltpu.sync_copy(ones_hbm.at[pl.ds(0, CHUNK)], ones_v)
    pltpu.sync_copy(dst_hbm.at[wid], dst_v)
    plsc.subcore_barrier()

    @pl.loop(0, CHUNKS // 2)
    def _(k):
        j = k * 2
        pltpu.async_copy(ones_v, acc.at[dst_v.at[j]], s0, add=True)
        pltpu.async_copy(ones_v, acc.at[dst_v.at[j + 1]], s1, add=True)
        pltpu.make_async_copy(ones_v, acc.at[dst_v.at[j]], s0).wait()
        pltpu.make_async_copy(ones_v, acc.at[dst_v.at[j + 1]], s1).wait()

    pltpu.sync_copy(ones_v, acc.at[dst_v.at[CHUNKS - 1]], add=True)

    plsc.subcore_barrier()
    pltpu.sync_copy(acc.at[pl.ds(r0, TPW)], out_hbm.at[c].at[pl.ds(r0, TPW)])


_RB = 1280
_GRID = (N1 // _RB,)


def _rows_spec(D):
    return pl.BlockSpec((_RB, D), lambda i: (i, 0))


def _pair_spec(D):
    return pl.BlockSpec((NSC, _RB, D), lambda i: (0, i, 0))


def _full_spec(a, b):
    return pl.BlockSpec((a, b), lambda i: (0, 0))


def _dis_spec():
    return pl.BlockSpec((_RB, 1), lambda i: (i, 0))


def _dis_block(dis_ref):
    return dis_ref[...]


def _t0_body(d_ref, x_ref, dis_ref, hs_ref):
    i = pl.program_id(0)
    deg = d_ref[0] + d_ref[1] - 1.0
    row = i * _RB + lax.broadcasted_iota(jnp.int32, (_RB, 1), 0)
    dis = jnp.where(row < N, lax.rsqrt(deg), 0.0)
    dis_ref[...] = dis
    hs_ref[...] = x_ref[...] * dis


def _t0(d, x_pad):
    return pl.pallas_call(
        _t0_body,
        grid=_GRID,
        in_specs=[_pair_spec(1), _rows_spec(F)],
        out_specs=[_dis_spec(), _rows_spec(F)],
        out_shape=[
            jax.ShapeDtypeStruct((N1, 1), jnp.float32),
            jax.ShapeDtypeStruct((N1, F), jnp.float32),
        ],
    )(d, x_pad)


def _t1_body(s_ref, hs_ref, dis_ref, w1_ref, b1_ref, w2_ref, out_ref):
    dis = _dis_block(dis_ref)
    ax = (s_ref[0] + s_ref[1] - hs_ref[...]) * dis
    h1 = jax.nn.relu(
        jnp.dot(ax, w1_ref[...], preferred_element_type=jnp.float32)
        + b1_ref[...]
    )
    out_ref[...] = (
        jnp.dot(h1, w2_ref[...], preferred_element_type=jnp.float32) * dis
    )


def _t1(s, hs0, dis, W1, b1, W2):
    return pl.pallas_call(
        _t1_body,
        grid=_GRID,
        in_specs=[
            _pair_spec(F),
            _rows_spec(F),
            _dis_spec(),
            _full_spec(F, 3 * H),
            _full_spec(1, 3 * H),
            _full_spec(3 * H, 2 * H),
        ],
        out_specs=_rows_spec(2 * H),
        out_shape=jax.ShapeDtypeStruct((N1, 2 * H), jnp.float32),
    )(s, hs0, dis, W1, b1.reshape(1, -1), W2)


def _mid_body(s_ref, hs_ref, dis_ref, b_ref, w_ref, out_ref):
    dis = _dis_block(dis_ref)
    h = jax.nn.relu((s_ref[0] + s_ref[1] - hs_ref[...]) * dis + b_ref[...])
    out_ref[...] = (
        jnp.dot(h, w_ref[...], preferred_element_type=jnp.float32) * dis
    )


def _t_mid(s, hs, dis, b, Wn, Din, Dout):
    return pl.pallas_call(
        _mid_body,
        grid=_GRID,
        in_specs=[
            _pair_spec(Din),
            _rows_spec(Din),
            _dis_spec(),
            _full_spec(1, Din),
            _full_spec(Din, Dout),
        ],
        out_specs=_rows_spec(Dout),
        out_shape=jax.ShapeDtypeStruct((N1, Dout), jnp.float32),
    )(s, hs, dis, b.reshape(1, -1), Wn)


def _t4_body(s_ref, hs_ref, dis_ref, b_ref, out_ref):
    dis = _dis_block(dis_ref)
    out_ref[...] = (
        (s_ref[0] + s_ref[1] - hs_ref[...]) * dis + b_ref[...]
    )


def _t4(s, hs, dis, b):
    return pl.pallas_call(
        _t4_body,
        grid=_GRID,
        in_specs=[
            _pair_spec(C),
            _rows_spec(C),
            _dis_spec(),
            _full_spec(1, C),
        ],
        out_specs=_rows_spec(C),
        out_shape=jax.ShapeDtypeStruct((N1, C), jnp.float32),
    )(s, hs, dis, b.reshape(1, -1))


_prop128 = _make_prop(F)
_prop64 = _make_prop(H)
_prop16 = _make_prop(C)


def kernel(x, edge_index, W1, b1, W2, b2, W3, b3, W4, b4):
    src = edge_index[0]
    dst = edge_index[1]

    x_pad = jnp.pad(x, ((0, N1 - N), (0, 0)))
    ones = jnp.ones((N1, 1), jnp.float32)

    d = _deg_pass(ones, dst.reshape(NW, CHUNKS, CHUNK))
    dis, hs0 = _t0(d, x_pad)

    s1 = _prop128(hs0, src, dst)
    hs1 = _t1(s1, hs0, dis, W1, b1, W2)

    s2 = _prop128(hs1, src, dst)
    hs2 = _t_mid(s2, hs1, dis, b2, W3, 2 * H, H)

    s3 = _prop64(hs2, src, dst)
    hs3 = _t_mid(s3, hs2, dis, b3, W4, H, C)

    s4 = _prop16(hs3, src, dst)
    out = _t4(s4, hs3, dis, b4)
    return out[:N]
